# baseline (speedup 1.0000x reference)
import jax
import jax.numpy as jnp
from jax.experimental import pallas as pl
from jax.experimental.pallas import tpu as pltpu

_COLS = 1024
_ROWS_PER_BLOCK = 128


def _body(yp_ref, s_ref, g_ref, out_ref, s1_ref, sp_ref, n1_ref, np_ref):
    i = pl.program_id(0)
    k = pl.num_programs(0)

    yp = yp_ref[...]
    sv = s_ref[...]
    gv = g_ref[...]
    pos = gv == 1
    m1 = pos & (sv == 1)
    zero = jnp.float32(0.0)
    posf = jnp.where(pos, yp, zero)
    m1f = jnp.where(m1, yp, zero)
    posc = jnp.where(pos, jnp.int32(1), jnp.int32(0))
    m1c = jnp.where(m1, jnp.int32(1), jnp.int32(0))

    @pl.when(i == 0)
    def _init():
        s1_ref[...] = m1f
        sp_ref[...] = posf
        n1_ref[...] = m1c
        np_ref[...] = posc

    @pl.when(i > 0)
    def _acc():
        s1_ref[...] += m1f
        sp_ref[...] += posf
        n1_ref[...] += m1c
        np_ref[...] += posc

    @pl.when(i == k - 1)
    def _fini():
        sum1 = jnp.sum(s1_ref[...])
        sumpos = jnp.sum(sp_ref[...])
        n1 = jnp.sum(n1_ref[...]).astype(jnp.float32)
        npos = jnp.sum(np_ref[...]).astype(jnp.float32)
        sum0 = sumpos - sum1
        n0 = npos - n1
        mean0 = sum0 / jnp.maximum(n0, jnp.float32(1.0))
        mean1 = sum1 / jnp.maximum(n1, jnp.float32(1.0))
        loss = jnp.abs(mean0 - mean1)
        out_ref[0] = jnp.where((n0 == 0.0) | (n1 == 0.0), jnp.float32(0.0), loss)


def kernel(y_pred, s, y_gt):
    n = y_pred.size
    rows = n // _COLS
    grid = rows // _ROWS_PER_BLOCK
    yp = y_pred.reshape(rows, _COLS)
    sv = s.astype(jnp.int32).reshape(rows, _COLS)
    gv = y_gt.astype(jnp.int32).reshape(rows, _COLS)

    blk = (_ROWS_PER_BLOCK, _COLS)
    in_spec = pl.BlockSpec(blk, lambda i: (i, 0))
    out = pl.pallas_call(
        _body,
        grid=(grid,),
        in_specs=[in_spec, in_spec, in_spec],
        out_specs=pl.BlockSpec(memory_space=pltpu.SMEM),
        out_shape=jax.ShapeDtypeStruct((1,), jnp.float32),
        scratch_shapes=[
            pltpu.VMEM(blk, jnp.float32),
            pltpu.VMEM(blk, jnp.float32),
            pltpu.VMEM(blk, jnp.int32),
            pltpu.VMEM(blk, jnp.int32),
        ],
    )(yp, sv, gv)
    return out[0]


# speedup vs baseline: 3.9952x; 3.9952x over previous
import jax
import jax.numpy as jnp
from jax.experimental import pallas as pl
from jax.experimental.pallas import tpu as pltpu

_COLS = 128
_ROWS_PER_BLOCK = 2048


def _body(yp_ref, s_ref, g_ref, out_ref, s1_ref, sp_ref, n1_ref, np_ref):
    i = pl.program_id(0)
    k = pl.num_programs(0)

    yp = yp_ref[...]
    gv = g_ref[...]
    gs = gv & s_ref[...]
    gf = gv.astype(jnp.float32)
    gsf = gs.astype(jnp.float32)
    posf = gf * yp
    m1f = gsf * yp

    def chunk_sum(x):
        return jnp.sum(x.reshape(-1, 8, 128), axis=0)

    p_sp = chunk_sum(posf)
    p_s1 = chunk_sum(m1f)
    p_np = chunk_sum(gf)
    p_n1 = chunk_sum(gsf)

    @pl.when(i == 0)
    def _init():
        sp_ref[...] = p_sp
        s1_ref[...] = p_s1
        np_ref[...] = p_np
        n1_ref[...] = p_n1

    @pl.when(i > 0)
    def _acc():
        sp_ref[...] += p_sp
        s1_ref[...] += p_s1
        np_ref[...] += p_np
        n1_ref[...] += p_n1

    @pl.when(i == k - 1)
    def _fini():
        sum1 = jnp.sum(s1_ref[...])
        sumpos = jnp.sum(sp_ref[...])
        n1 = jnp.sum(n1_ref[...])
        npos = jnp.sum(np_ref[...])
        sum0 = sumpos - sum1
        n0 = npos - n1
        mean0 = sum0 / jnp.maximum(n0, jnp.float32(1.0))
        mean1 = sum1 / jnp.maximum(n1, jnp.float32(1.0))
        loss = jnp.abs(mean0 - mean1)
        out_ref[0] = jnp.where((n0 == 0.0) | (n1 == 0.0), jnp.float32(0.0), loss)


def kernel(y_pred, s, y_gt):
    n = y_pred.size
    rows = n // _COLS
    grid = rows // _ROWS_PER_BLOCK
    yp = y_pred.reshape(rows, _COLS)
    sv = s.astype(jnp.int32).reshape(rows, _COLS)
    gv = y_gt.astype(jnp.int32).reshape(rows, _COLS)

    blk = (_ROWS_PER_BLOCK, _COLS)
    in_spec = pl.BlockSpec(blk, lambda i: (i, 0))
    out = pl.pallas_call(
        _body,
        grid=(grid,),
        in_specs=[in_spec, in_spec, in_spec],
        out_specs=pl.BlockSpec(memory_space=pltpu.SMEM),
        out_shape=jax.ShapeDtypeStruct((1,), jnp.float32),
        scratch_shapes=[
            pltpu.VMEM((8, 128), jnp.float32),
            pltpu.VMEM((8, 128), jnp.float32),
            pltpu.VMEM((8, 128), jnp.float32),
            pltpu.VMEM((8, 128), jnp.float32),
        ],
    )(yp, sv, gv)
    return out[0]


# speedup vs baseline: 4.7207x; 1.1816x over previous
import jax
import jax.numpy as jnp
from jax.experimental import pallas as pl
from jax.experimental.pallas import tpu as pltpu

_COLS = 128
_ROWS_PER_BLOCK = 4096


def _body(yp_ref, s_ref, g_ref, out_ref, s1_ref, sp_ref, n1_ref, np_ref):
    i = pl.program_id(0)
    k = pl.num_programs(0)

    yp = yp_ref[...]
    gv = g_ref[...]
    gs = gv & s_ref[...]
    gf = gv.astype(jnp.float32)
    gsf = gs.astype(jnp.float32)
    posf = gf * yp
    m1f = gsf * yp

    def chunk_sum(x):
        return jnp.sum(x.reshape(-1, 8, 128), axis=0)

    p_sp = chunk_sum(posf)
    p_s1 = chunk_sum(m1f)
    p_np = chunk_sum(gf)
    p_n1 = chunk_sum(gsf)

    @pl.when(i == 0)
    def _init():
        sp_ref[...] = p_sp
        s1_ref[...] = p_s1
        np_ref[...] = p_np
        n1_ref[...] = p_n1

    @pl.when(i > 0)
    def _acc():
        sp_ref[...] += p_sp
        s1_ref[...] += p_s1
        np_ref[...] += p_np
        n1_ref[...] += p_n1

    @pl.when(i == k - 1)
    def _fini():
        sum1 = jnp.sum(s1_ref[...])
        sumpos = jnp.sum(sp_ref[...])
        n1 = jnp.sum(n1_ref[...])
        npos = jnp.sum(np_ref[...])
        sum0 = sumpos - sum1
        n0 = npos - n1
        mean0 = sum0 / jnp.maximum(n0, jnp.float32(1.0))
        mean1 = sum1 / jnp.maximum(n1, jnp.float32(1.0))
        loss = jnp.abs(mean0 - mean1)
        out_ref[0] = jnp.where((n0 == 0.0) | (n1 == 0.0), jnp.float32(0.0), loss)


def kernel(y_pred, s, y_gt):
    n = y_pred.size
    rows = n // _COLS
    grid = rows // _ROWS_PER_BLOCK
    yp = y_pred.reshape(rows, _COLS)
    sv = s.astype(jnp.int32).reshape(rows, _COLS)
    gv = y_gt.astype(jnp.int32).reshape(rows, _COLS)

    blk = (_ROWS_PER_BLOCK, _COLS)
    in_spec = pl.BlockSpec(blk, lambda i: (i, 0))
    out = pl.pallas_call(
        _body,
        grid=(grid,),
        in_specs=[in_spec, in_spec, in_spec],
        out_specs=pl.BlockSpec(memory_space=pltpu.SMEM),
        out_shape=jax.ShapeDtypeStruct((1,), jnp.float32),
        scratch_shapes=[
            pltpu.VMEM((8, 128), jnp.float32),
            pltpu.VMEM((8, 128), jnp.float32),
            pltpu.VMEM((8, 128), jnp.float32),
            pltpu.VMEM((8, 128), jnp.float32),
        ],
    )(yp, sv, gv)
    return out[0]


# speedup vs baseline: 4.7376x; 1.0036x over previous
import jax
import jax.numpy as jnp
from jax.experimental import pallas as pl
from jax.experimental.pallas import tpu as pltpu

_COLS = 128
_ROWS_PER_BLOCK = 8192


def _body(yp_ref, s_ref, g_ref, out_ref, s1_ref, sp_ref, n1_ref, np_ref):
    i = pl.program_id(0)
    k = pl.num_programs(0)

    yp = yp_ref[...]
    gv = g_ref[...]
    gs = gv & s_ref[...]
    gf = gv.astype(jnp.float32)
    gsf = gs.astype(jnp.float32)
    posf = gf * yp
    m1f = gsf * yp

    def chunk_sum(x):
        return jnp.sum(x.reshape(-1, 8, 128), axis=0)

    p_sp = chunk_sum(posf)
    p_s1 = chunk_sum(m1f)
    p_np = chunk_sum(gf)
    p_n1 = chunk_sum(gsf)

    @pl.when(i == 0)
    def _init():
        sp_ref[...] = p_sp
        s1_ref[...] = p_s1
        np_ref[...] = p_np
        n1_ref[...] = p_n1

    @pl.when(i > 0)
    def _acc():
        sp_ref[...] += p_sp
        s1_ref[...] += p_s1
        np_ref[...] += p_np
        n1_ref[...] += p_n1

    @pl.when(i == k - 1)
    def _fini():
        sum1 = jnp.sum(s1_ref[...])
        sumpos = jnp.sum(sp_ref[...])
        n1 = jnp.sum(n1_ref[...])
        npos = jnp.sum(np_ref[...])
        sum0 = sumpos - sum1
        n0 = npos - n1
        mean0 = sum0 / jnp.maximum(n0, jnp.float32(1.0))
        mean1 = sum1 / jnp.maximum(n1, jnp.float32(1.0))
        loss = jnp.abs(mean0 - mean1)
        out_ref[0] = jnp.where((n0 == 0.0) | (n1 == 0.0), jnp.float32(0.0), loss)


def kernel(y_pred, s, y_gt):
    n = y_pred.size
    rows = n // _COLS
    grid = rows // _ROWS_PER_BLOCK
    yp = y_pred.reshape(rows, _COLS)
    sv = s.astype(jnp.int32).reshape(rows, _COLS)
    gv = y_gt.astype(jnp.int32).reshape(rows, _COLS)

    blk = (_ROWS_PER_BLOCK, _COLS)
    in_spec = pl.BlockSpec(blk, lambda i: (i, 0))
    out = pl.pallas_call(
        _body,
        grid=(grid,),
        in_specs=[in_spec, in_spec, in_spec],
        out_specs=pl.BlockSpec(memory_space=pltpu.SMEM),
        out_shape=jax.ShapeDtypeStruct((1,), jnp.float32),
        scratch_shapes=[
            pltpu.VMEM((8, 128), jnp.float32),
            pltpu.VMEM((8, 128), jnp.float32),
            pltpu.VMEM((8, 128), jnp.float32),
            pltpu.VMEM((8, 128), jnp.float32),
        ],
    )(yp, sv, gv)
    return out[0]
